# baseline (speedup 1.0000x reference)
import jax
import jax.numpy as jnp
from jax import lax
from jax.experimental import pallas as pl
from jax.experimental.pallas import tpu as pltpu
from jax.experimental.pallas import tpu_sc as plsc

_ALPHA = 4
_C, _T, _H, _W = 3, 64, 512, 512
_NSLOW = _T // _ALPHA
assert [(63 * j) // 15 for j in range(_NSLOW)] == [
    0, 4, 8, 12, 16, 21, 25, 29, 33, 37, 42, 46, 50, 54, 58, 63]

_NSLABS_IN = _C * _T
_NSLABS_OUT = _C * _NSLOW
_CHUNK = 64
_CPS = _H // _CHUNK
_NCHUNKS = _NSLABS_OUT * _CPS


def _slow_gather(slabs):
    info = plsc.get_sparse_core_info()
    nw = info.num_cores * info.num_subcores
    assert _NCHUNKS % nw == 0
    nb = _NCHUNKS // nw
    nbuf = 3
    mesh = plsc.VectorSubcoreMesh(core_axis_name="c", subcore_axis_name="s")

    def body(in_hbm, out_hbm, *rest):
        w = lax.axis_index("c") * info.num_subcores + lax.axis_index("s")
        bufs = rest[0:nbuf]
        isems = rest[nbuf:2 * nbuf]
        osems = rest[2 * nbuf:3 * nbuf]

        def start_in(b):
            t = w * nb + b
            f = t // _CPS
            r = (t % _CPS) * _CHUNK
            ch = f // _NSLOW
            j = f % _NSLOW
            src_slab = ch * _T + (63 * j) // 15
            c = pltpu.make_async_copy(
                in_hbm.at[src_slab, pl.ds(r, _CHUNK), :],
                bufs[b % nbuf], isems[b % nbuf])
            c.start()
            return c

        def start_out(b):
            t = w * nb + b
            c = pltpu.make_async_copy(
                bufs[b % nbuf],
                out_hbm.at[t // _CPS, pl.ds((t % _CPS) * _CHUNK, _CHUNK), :],
                osems[b % nbuf])
            c.start()
            return c

        cin = [None] * nb
        cout = [None] * nb
        for g in range(nbuf - 1):
            cin[g] = start_in(g)
        for b in range(nb):
            nxt = b + nbuf - 1
            if nxt < nb:
                if b >= 1:
                    cout[b - 1].wait()
                cin[nxt] = start_in(nxt)
            cin[b].wait()
            cout[b] = start_out(b)
        for b in range(max(0, nb - nbuf), nb):
            cout[b].wait()

    run = pl.kernel(
        body,
        out_type=jax.ShapeDtypeStruct((_NSLABS_OUT, _H, _W), jnp.float32),
        mesh=mesh,
        scratch_types=(
            [pltpu.VMEM((_CHUNK, _W), jnp.float32)] * 3
            + [pltpu.SemaphoreType.DMA] * 6),
    )
    return run(slabs)


def _fast_copy(slabs):
    per = 2
    nch = _NSLABS_IN // per
    nbuf = 6

    def body(in_ref, out_ref, *rest):
        bufs = rest[0:nbuf]
        isems = rest[nbuf:2 * nbuf]
        osems = rest[2 * nbuf:3 * nbuf]

        def sin(g):
            c = pltpu.make_async_copy(
                in_ref.at[pl.ds(g * per, per)], bufs[g % nbuf],
                isems[g % nbuf])
            c.start()
            return c

        def sout(g):
            c = pltpu.make_async_copy(
                bufs[g % nbuf], out_ref.at[pl.ds(g * per, per)],
                osems[g % nbuf])
            c.start()
            return c

        cin = [None] * nch
        cout = [None] * nch
        for g in range(nbuf - 1):
            cin[g] = sin(g)
        for g in range(nch):
            nxt = g + nbuf - 1
            if nxt < nch:
                if g >= 1:
                    cout[g - 1].wait()
                cin[nxt] = sin(nxt)
            cin[g].wait()
            cout[g] = sout(g)
        for g in range(max(0, nch - nbuf), nch):
            cout[g].wait()

    return pl.pallas_call(
        body,
        in_specs=[pl.BlockSpec(memory_space=pl.ANY)],
        out_specs=pl.BlockSpec(memory_space=pl.ANY),
        out_shape=jax.ShapeDtypeStruct((_NSLABS_IN, _H, _W), jnp.float32),
        scratch_shapes=(
            [pltpu.VMEM((per, _H, _W), jnp.float32)] * nbuf
            + [pltpu.SemaphoreType.DMA] * (2 * nbuf)),
    )(slabs)


def kernel(frames):
    slabs = frames.reshape(_NSLABS_IN, _H, _W)
    slow = _slow_gather(slabs).reshape(_C, _NSLOW, _H, _W)
    fast = _fast_copy(slabs).reshape(_C, _T, _H, _W)
    return (slow, fast)


# speedup vs baseline: 1.0600x; 1.0600x over previous
import jax
import jax.numpy as jnp
from jax import lax
from jax.experimental import pallas as pl
from jax.experimental.pallas import tpu as pltpu
from jax.experimental.pallas import tpu_sc as plsc

_ALPHA = 4
_C, _T, _H, _W = 3, 64, 512, 512
_NSLOW = _T // _ALPHA
assert [int((_T - 1) * j // (_NSLOW - 1)) for j in range(_NSLOW)] == [
    0, 4, 8, 12, 16, 21, 25, 29, 33, 37, 42, 46, 50, 54, 58, 63]

_NSLABS_IN = _C * _T
_NSLABS_OUT = _C * _NSLOW
_CHUNK = 64
_CPS = _H // _CHUNK
_NCHUNKS = _NSLABS_OUT * _CPS


def _slow_gather(flat):
    info = plsc.get_sparse_core_info()
    nw = info.num_cores * info.num_subcores
    assert _NCHUNKS % nw == 0
    nb = _NCHUNKS // nw
    mesh = plsc.VectorSubcoreMesh(core_axis_name="c", subcore_axis_name="s")

    def body(in_hbm, out_hbm, buf0, buf1, isem0, isem1, osem0, osem1):
        w = lax.axis_index("c") * info.num_subcores + lax.axis_index("s")
        bufs = (buf0, buf1)
        isems = (isem0, isem1)
        osems = (osem0, osem1)

        def start_in(b):
            t = w * nb + b
            f = t // _CPS
            r = (t % _CPS) * _CHUNK
            ch = f // _NSLOW
            j = f % _NSLOW
            src_slab = ch * _T + ((_T - 1) * j) // (_NSLOW - 1)
            c = pltpu.make_async_copy(
                in_hbm.at[src_slab, pl.ds(r, _CHUNK), :],
                bufs[b % 2], isems[b % 2])
            c.start()
            return c

        def start_out(b):
            t = w * nb + b
            c = pltpu.make_async_copy(
                bufs[b % 2],
                out_hbm.at[t // _CPS, pl.ds((t % _CPS) * _CHUNK, _CHUNK), :],
                osems[b % 2])
            c.start()
            return c

        cin = [None] * nb
        cout = [None] * nb
        cin[0] = start_in(0)
        for b in range(nb):
            if b + 1 < nb:
                if b >= 1:
                    cout[b - 1].wait()
                cin[b + 1] = start_in(b + 1)
            cin[b].wait()
            cout[b] = start_out(b)
        cout[nb - 2].wait()
        cout[nb - 1].wait()

    run = pl.kernel(
        body,
        out_type=jax.ShapeDtypeStruct((_NSLABS_OUT, _H, _W), jnp.float32),
        mesh=mesh,
        scratch_types=[
            pltpu.VMEM((_CHUNK, _W), jnp.float32),
            pltpu.VMEM((_CHUNK, _W), jnp.float32),
            pltpu.SemaphoreType.DMA,
            pltpu.SemaphoreType.DMA,
            pltpu.SemaphoreType.DMA,
            pltpu.SemaphoreType.DMA,
        ],
    )
    return run(flat)


def _fast_copy(slabs):
    per = 8
    nch = _NSLABS_IN // per
    nbuf = 4

    def body(in_ref, out_ref, *rest):
        bufs = rest[:nbuf]
        isems = rest[nbuf:2 * nbuf]
        osems = rest[2 * nbuf:]

        def sin(g):
            c = pltpu.make_async_copy(
                in_ref.at[pl.ds(g * per, per)], bufs[g % nbuf],
                isems[g % nbuf])
            c.start()
            return c

        def sout(g):
            c = pltpu.make_async_copy(
                bufs[g % nbuf], out_ref.at[pl.ds(g * per, per)],
                osems[g % nbuf])
            c.start()
            return c

        cin = [None] * nch
        cout = [None] * nch
        for g in range(nbuf):
            cin[g] = sin(g)
        for g in range(nch):
            cin[g].wait()
            cout[g] = sout(g)
            if g + nbuf < nch:
                cout[g].wait()
                cin[g + nbuf] = sin(g + nbuf)
        for g in range(max(0, nch - nbuf), nch):
            cout[g].wait()

    return pl.pallas_call(
        body,
        in_specs=[pl.BlockSpec(memory_space=pl.ANY)],
        out_specs=pl.BlockSpec(memory_space=pl.ANY),
        out_shape=jax.ShapeDtypeStruct((_NSLABS_IN, _H, _W), jnp.float32),
        scratch_shapes=(
            [pltpu.VMEM((per, _H, _W), jnp.float32)] * nbuf
            + [pltpu.SemaphoreType.DMA] * (2 * nbuf)),
    )(slabs)


def kernel(frames):
    slabs = frames.reshape(_NSLABS_IN, _H, _W)
    slow = _slow_gather(slabs).reshape(_C, _NSLOW, _H, _W)
    fast = _fast_copy(slabs).reshape(_C, _T, _H, _W)
    return (slow, fast)


# speedup vs baseline: 1.0673x; 1.0069x over previous
import jax
import jax.numpy as jnp
from jax import lax
from jax.experimental import pallas as pl
from jax.experimental.pallas import tpu as pltpu
from jax.experimental.pallas import tpu_sc as plsc

_ALPHA = 4
_C, _T, _H, _W = 3, 64, 512, 512
_NSLOW = _T // _ALPHA
assert [int((_T - 1) * j // (_NSLOW - 1)) for j in range(_NSLOW)] == [
    0, 4, 8, 12, 16, 21, 25, 29, 33, 37, 42, 46, 50, 54, 58, 63]

_NSLABS_IN = _C * _T
_NSLABS_OUT = _C * _NSLOW
_CHUNK = 64
_CPS = _H // _CHUNK
_NCHUNKS = _NSLABS_OUT * _CPS


def _slow_gather(flat):
    info = plsc.get_sparse_core_info()
    nw = info.num_cores * info.num_subcores
    assert _NCHUNKS % nw == 0
    nb = _NCHUNKS // nw
    mesh = plsc.VectorSubcoreMesh(core_axis_name="c", subcore_axis_name="s")

    def body(in_hbm, out_hbm, buf0, buf1, isem0, isem1, osem0, osem1):
        w = lax.axis_index("c") * info.num_subcores + lax.axis_index("s")
        bufs = (buf0, buf1)
        isems = (isem0, isem1)
        osems = (osem0, osem1)

        def start_in(b):
            t = w * nb + b
            f = t // _CPS
            r = (t % _CPS) * _CHUNK
            ch = f // _NSLOW
            j = f % _NSLOW
            src_slab = ch * _T + ((_T - 1) * j) // (_NSLOW - 1)
            c = pltpu.make_async_copy(
                in_hbm.at[src_slab, pl.ds(r, _CHUNK), :],
                bufs[b % 2], isems[b % 2])
            c.start()
            return c

        def start_out(b):
            t = w * nb + b
            c = pltpu.make_async_copy(
                bufs[b % 2],
                out_hbm.at[t // _CPS, pl.ds((t % _CPS) * _CHUNK, _CHUNK), :],
                osems[b % 2])
            c.start()
            return c

        cin = [None] * nb
        cout = [None] * nb
        cin[0] = start_in(0)
        for b in range(nb):
            if b + 1 < nb:
                if b >= 1:
                    cout[b - 1].wait()
                cin[b + 1] = start_in(b + 1)
            cin[b].wait()
            cout[b] = start_out(b)
        cout[nb - 2].wait()
        cout[nb - 1].wait()

    run = pl.kernel(
        body,
        out_type=jax.ShapeDtypeStruct((_NSLABS_OUT, _H, _W), jnp.float32),
        mesh=mesh,
        scratch_types=[
            pltpu.VMEM((_CHUNK, _W), jnp.float32),
            pltpu.VMEM((_CHUNK, _W), jnp.float32),
            pltpu.SemaphoreType.DMA,
            pltpu.SemaphoreType.DMA,
            pltpu.SemaphoreType.DMA,
            pltpu.SemaphoreType.DMA,
        ],
    )
    return run(flat)


def _fast_copy(slabs):
    per = 12
    nch = _NSLABS_IN // per
    nbuf = 4

    def body(in_ref, out_ref, *rest):
        bufs = rest[:nbuf]
        isems = rest[nbuf:2 * nbuf]
        osems = rest[2 * nbuf:]

        def sin(g):
            c = pltpu.make_async_copy(
                in_ref.at[pl.ds(g * per, per)], bufs[g % nbuf],
                isems[g % nbuf])
            c.start()
            return c

        def sout(g):
            c = pltpu.make_async_copy(
                bufs[g % nbuf], out_ref.at[pl.ds(g * per, per)],
                osems[g % nbuf])
            c.start()
            return c

        cin = [None] * nch
        cout = [None] * nch
        for g in range(nbuf):
            cin[g] = sin(g)
        for g in range(nch):
            cin[g].wait()
            cout[g] = sout(g)
            if g + nbuf < nch:
                cout[g].wait()
                cin[g + nbuf] = sin(g + nbuf)
        for g in range(max(0, nch - nbuf), nch):
            cout[g].wait()

    return pl.pallas_call(
        body,
        in_specs=[pl.BlockSpec(memory_space=pl.ANY)],
        out_specs=pl.BlockSpec(memory_space=pl.ANY),
        out_shape=jax.ShapeDtypeStruct((_NSLABS_IN, _H, _W), jnp.float32),
        scratch_shapes=(
            [pltpu.VMEM((per, _H, _W), jnp.float32)] * nbuf
            + [pltpu.SemaphoreType.DMA] * (2 * nbuf)),
    )(slabs)


def kernel(frames):
    slabs = frames.reshape(_NSLABS_IN, _H, _W)
    slow = _slow_gather(slabs).reshape(_C, _NSLOW, _H, _W)
    fast = _fast_copy(slabs).reshape(_C, _T, _H, _W)
    return (slow, fast)
